# Initial kernel scaffold; baseline (speedup 1.0000x reference)
import jax
import jax.numpy as jnp
from jax.experimental import pallas as pl


def kernel(src_node_features, neighbor_node_features, W_agg, W):
    raise NotImplementedError("write your pallas kernel here")


# speedup vs baseline: 1.2227x; 1.2227x over previous
import jax
import jax.numpy as jnp
from jax.experimental import pallas as pl
from jax.experimental.pallas import tpu as pltpu

NUM_SRC = 10000
NUM_NEIGH = 16
DIM = 256
BLOCK_M = 400


def _fused_body(src_ref, neigh_ref, w_agg_ref, w_ref, out_ref):
    aggr = jnp.sum(neigh_ref[...], axis=1) * (1.0 / NUM_NEIGH)
    h = jnp.dot(aggr, w_agg_ref[...], preferred_element_type=jnp.float32)
    h = h + src_ref[...]
    out = jnp.dot(h, w_ref[...], preferred_element_type=jnp.float32)
    out_ref[...] = jnp.maximum(out, 0.0)


def kernel(src_node_features, nei_node_features, W_agg, W):
    n = src_node_features.shape[0]
    grid = (n // BLOCK_M,)
    return pl.pallas_call(
        _fused_body,
        grid=grid,
        in_specs=[
            pl.BlockSpec((BLOCK_M, DIM), lambda i: (i, 0)),
            pl.BlockSpec((BLOCK_M, NUM_NEIGH, DIM), lambda i: (i, 0, 0)),
            pl.BlockSpec((DIM, DIM), lambda i: (0, 0)),
            pl.BlockSpec((DIM, DIM), lambda i: (0, 0)),
        ],
        out_specs=pl.BlockSpec((BLOCK_M, DIM), lambda i: (i, 0)),
        out_shape=jax.ShapeDtypeStruct((n, DIM), jnp.float32),
        compiler_params=pltpu.CompilerParams(
            dimension_semantics=("arbitrary",),
        ),
    )(src_node_features, nei_node_features, W_agg, W)


# speedup vs baseline: 1.2559x; 1.0271x over previous
import jax
import jax.numpy as jnp
from jax.experimental import pallas as pl
from jax.experimental.pallas import tpu as pltpu

NUM_SRC = 10000
NUM_NEIGH = 16
DIM = 256
BLOCK_M = 1000


def _fused_body(src_ref, neigh_ref, w_agg_ref, w_ref, out_ref):
    aggr = jnp.sum(neigh_ref[...], axis=1) * (1.0 / NUM_NEIGH)
    h = jnp.dot(aggr, w_agg_ref[...], preferred_element_type=jnp.float32)
    h = h + src_ref[...]
    out = jnp.dot(h, w_ref[...], preferred_element_type=jnp.float32)
    out_ref[...] = jnp.maximum(out, 0.0)


def kernel(src_node_features, nei_node_features, W_agg, W):
    n = src_node_features.shape[0]
    grid = (n // BLOCK_M,)
    return pl.pallas_call(
        _fused_body,
        grid=grid,
        in_specs=[
            pl.BlockSpec((BLOCK_M, DIM), lambda i: (i, 0)),
            pl.BlockSpec((BLOCK_M, NUM_NEIGH, DIM), lambda i: (i, 0, 0)),
            pl.BlockSpec((DIM, DIM), lambda i: (0, 0)),
            pl.BlockSpec((DIM, DIM), lambda i: (0, 0)),
        ],
        out_specs=pl.BlockSpec((BLOCK_M, DIM), lambda i: (i, 0)),
        out_shape=jax.ShapeDtypeStruct((n, DIM), jnp.float32),
        compiler_params=pltpu.CompilerParams(
            dimension_semantics=("arbitrary",),
        ),
    )(src_node_features, nei_node_features, W_agg, W)
